# Initial kernel scaffold; baseline (speedup 1.0000x reference)
import jax
import jax.numpy as jnp
from jax.experimental import pallas as pl


def kernel(cat_idx, sub_idx, elem_idx, edge_index, emb_cat, emb_sub, emb_elem, W1, b1, W2, b2):
    raise NotImplementedError("write your pallas kernel here")


# speedup vs baseline: 9.7469x; 9.7469x over previous
import functools

import jax
import jax.numpy as jnp
from jax import lax
from jax.experimental import pallas as pl
from jax.experimental.pallas import tpu as pltpu
from jax.experimental.pallas import tpu_sc as plsc

N = 50000
E = 800000
NP = 51200
EP = 819200
CW = 32
BN = 40
BE = 128
NC, NS = 2, 16
RW = NP // (NC * NS)
RT = NP // NS
EPB = EP // BE
ETR = EP // NS // BE
EHR = EP // (NC * NS) // BE
IB = 40
ZR = 200

_mesh = plsc.VectorSubcoreMesh(core_axis_name="c", subcore_axis_name="s")


@functools.partial(
    pl.kernel,
    out_type=(
        jax.ShapeDtypeStruct((NP, CW), jnp.float32),
        jax.ShapeDtypeStruct((NP, CW), jnp.float32),
        jax.ShapeDtypeStruct((NP, CW), jnp.float32),
        jax.ShapeDtypeStruct((NC * NP,), jnp.float32),
    ),
    mesh=_mesh,
    compiler_params=pltpu.CompilerParams(use_tc_tiling_on_sc=False),
    scratch_types=[
        pltpu.VMEM((RW // BN, BN), jnp.int32),
        pltpu.VMEM((RW, CW), jnp.float32),
        pltpu.VMEM((EHR, BE), jnp.int32),
        pltpu.VMEM((BE,), jnp.float32),
        pltpu.VMEM_SHARED((NP,), jnp.float32),
        pltpu.SemaphoreType.DMA,
    ],
)
def _embed_deg(cat_h, sub_h, elem_h, dst_h, ones_h, zeros1_h,
               tcat_h, tsub_h, telem_h,
               xc_h, xs_h, xe_h, degp_h,
               idx_v, rows_v, eidx_v, ones_v, deg_acc, sem):
    cid = lax.axis_index("c")
    tid = lax.axis_index("s")
    wid = tid * NC + cid

    row0 = pl.multiple_of(tid * RT, 8)
    pltpu.sync_copy(zeros1_h, deg_acc.at[pl.ds(row0, RT)])
    pltpu.sync_copy(ones_h, ones_v)
    ebase = pl.multiple_of((cid * NS + tid) * EHR, 8)
    pltpu.sync_copy(dst_h.at[pl.ds(ebase, EHR), :], eidx_v)
    plsc.subcore_barrier()

    def dbody(j, c):
        pltpu.sync_copy(ones_v, deg_acc.at[eidx_v.at[j]], add=True)
        return c
    lax.fori_loop(0, EHR, dbody, 0)

    base = pl.multiple_of(wid * RW, 8)
    brow = pl.multiple_of(wid * (RW // BN), 8)
    for tab_h, nidx_h, out_h in ((tcat_h, cat_h, xc_h),
                                 (tsub_h, sub_h, xs_h),
                                 (telem_h, elem_h, xe_h)):
        pltpu.sync_copy(nidx_h.at[pl.ds(brow, RW // BN), :], idx_v)
        cps = [
            pltpu.async_copy(tab_h.at[idx_v.at[j]],
                             rows_v.at[pl.ds(j * BN, BN), :], sem)
            for j in range(RW // BN)
        ]
        for cp in cps:
            cp.wait()
        pltpu.sync_copy(rows_v, out_h.at[pl.ds(base, RW), :])

    plsc.subcore_barrier()
    orow = pl.multiple_of(cid * NP + tid * RT, 8)
    pltpu.sync_copy(deg_acc.at[pl.ds(row0, RT)], degp_h.at[pl.ds(orow, RT)])


def _make_agg(C):
    @functools.partial(
        pl.kernel,
        out_type=jax.ShapeDtypeStruct((C * NP, CW), jnp.float32),
        mesh=_mesh,
        compiler_params=pltpu.CompilerParams(use_tc_tiling_on_sc=False),
        scratch_types=[
            pltpu.VMEM((IB, BE), jnp.int32),
            pltpu.VMEM((IB, BE), jnp.int32),
            pltpu.VMEM((BE, CW), jnp.float32),
            pltpu.VMEM((ZR, CW), jnp.float32),
            pltpu.VMEM_SHARED((NP, CW), jnp.float32),
            pltpu.SemaphoreType.DMA,
        ],
    )
    def agg(y_h, srcadj_h, dst_h, zeros_h, s_h,
            src_v, dst_v, rows_v, zbuf, acc, sem):
        cid = lax.axis_index("c")
        tid = lax.axis_index("s")
        pltpu.sync_copy(zeros_h, zbuf)
        row0 = pl.multiple_of(tid * RT, 8)
        tb = pl.multiple_of(tid * ETR, 8)

        for k in range(C // 2):
            chunk = cid * (C // 2) + k
            for z in range(RT // ZR):
                pltpu.sync_copy(zbuf, acc.at[pl.ds(row0 + z * ZR, ZR), :])
            plsc.subcore_barrier()

            srow = pl.multiple_of(chunk * EPB + tb, 8)
            for blk in range(ETR // IB):
                pltpu.sync_copy(srcadj_h.at[pl.ds(srow + blk * IB, IB), :],
                                src_v)
                pltpu.sync_copy(dst_h.at[pl.ds(tb + blk * IB, IB), :], dst_v)

                def ebody(j, c):
                    pltpu.async_copy(y_h.at[src_v.at[j]], rows_v, sem).wait()
                    pltpu.sync_copy(rows_v, acc.at[dst_v.at[j]], add=True)
                    return c
                lax.fori_loop(0, IB, ebody, 0)
            plsc.subcore_barrier()

            orow = pl.multiple_of(chunk * NP + tid * RT, 8)
            pltpu.sync_copy(acc.at[pl.ds(row0, RT), :],
                            s_h.at[pl.ds(orow, RT), :])
    return agg


_agg4 = _make_agg(4)
_agg2 = _make_agg(2)

RB = 512


def _y1_body(xc, xs, xe, dg, w1, y1o):
    dinv = lax.rsqrt(dg[0, :] + dg[1, :] + 1.0)
    xw = (jnp.dot(xc[...], w1[0:32, :], preferred_element_type=jnp.float32)
          + jnp.dot(xs[...], w1[32:64, :], preferred_element_type=jnp.float32)
          + jnp.dot(xe[...], w1[64:96, :], preferred_element_type=jnp.float32))
    y = xw * dinv[:, None]
    for c in range(4):
        y1o[c] = y[:, c * CW:(c + 1) * CW]


def _y2_body(s1, y1, dg, b1, w2, y2o):
    dinv = lax.rsqrt(dg[0, :] + dg[1, :] + 1.0)
    acc = jnp.zeros((RB, 64), jnp.float32)
    for c in range(4):
        t = jnp.maximum((s1[c] + y1[c]) * dinv[:, None]
                        + b1[c * CW:(c + 1) * CW][None, :], 0.0)
        acc = acc + jnp.dot(t, w2[c * CW:(c + 1) * CW, :],
                            preferred_element_type=jnp.float32)
    y2 = acc * dinv[:, None]
    y2o[0] = y2[:, 0:CW]
    y2o[1] = y2[:, CW:2 * CW]


def _out_body(s2, y2, dg, b2, o):
    dinv = lax.rsqrt(dg[0, :] + dg[1, :] + 1.0)
    o0 = jnp.maximum((s2[0] + y2[0]) * dinv[:, None] + b2[0:CW][None, :], 0.0)
    o1 = jnp.maximum((s2[1] + y2[1]) * dinv[:, None] + b2[CW:2 * CW][None, :],
                     0.0)
    o[...] = jnp.concatenate([o0, o1], axis=1)


def kernel(cat_idx, sub_idx, elem_idx, edge_index,
           emb_cat, emb_sub, emb_elem, W1, b1, W2, b2):
    f32 = jnp.float32
    cat2 = jnp.pad(cat_idx, (0, NP - N)).reshape(NP // BN, BN)
    sub2 = jnp.pad(sub_idx, (0, NP - N)).reshape(NP // BN, BN)
    elem2 = jnp.pad(elem_idx, (0, NP - N)).reshape(NP // BN, BN)
    src_p = jnp.pad(edge_index[0], (0, EP - E), constant_values=NP - 1)
    dst_p = jnp.pad(edge_index[1], (0, EP - E), constant_values=NP - 1)
    dst2 = dst_p.reshape(EPB, BE)
    srcadj4 = (src_p[None, :]
               + (jnp.arange(4, dtype=jnp.int32) * NP)[:, None]
               ).reshape(4 * EPB, BE)
    srcadj2 = srcadj4[:2 * EPB]
    ones_be = jnp.ones((BE,), f32)
    zeros_rt = jnp.zeros((RT,), f32)
    zeros_zb = jnp.zeros((ZR, CW), f32)

    xc, xs, xe, degp = _embed_deg(cat2, sub2, elem2, dst2, ones_be, zeros_rt,
                                  emb_cat, emb_sub, emb_elem)
    degp2 = degp.reshape(2, NP)

    y1 = pl.pallas_call(
        _y1_body,
        grid=(NP // RB,),
        in_specs=[
            pl.BlockSpec((RB, CW), lambda i: (i, 0)),
            pl.BlockSpec((RB, CW), lambda i: (i, 0)),
            pl.BlockSpec((RB, CW), lambda i: (i, 0)),
            pl.BlockSpec((2, RB), lambda i: (0, i)),
            pl.BlockSpec((96, 128), lambda i: (0, 0)),
        ],
        out_specs=pl.BlockSpec((4, RB, CW), lambda i: (0, i, 0)),
        out_shape=jax.ShapeDtypeStruct((4, NP, CW), f32),
    )(xc, xs, xe, degp2, W1)

    s1 = _agg4(y1.reshape(4 * NP, CW), srcadj4, dst2, zeros_zb)

    y2 = pl.pallas_call(
        _y2_body,
        grid=(NP // RB,),
        in_specs=[
            pl.BlockSpec((4, RB, CW), lambda i: (0, i, 0)),
            pl.BlockSpec((4, RB, CW), lambda i: (0, i, 0)),
            pl.BlockSpec((2, RB), lambda i: (0, i)),
            pl.BlockSpec((128,), lambda i: (0,)),
            pl.BlockSpec((128, 64), lambda i: (0, 0)),
        ],
        out_specs=pl.BlockSpec((2, RB, CW), lambda i: (0, i, 0)),
        out_shape=jax.ShapeDtypeStruct((2, NP, CW), f32),
    )(s1.reshape(4, NP, CW), y1, degp2, b1, W2)

    s2 = _agg2(y2.reshape(2 * NP, CW), srcadj2, dst2, zeros_zb)

    out = pl.pallas_call(
        _out_body,
        grid=(NP // RB,),
        in_specs=[
            pl.BlockSpec((2, RB, CW), lambda i: (0, i, 0)),
            pl.BlockSpec((2, RB, CW), lambda i: (0, i, 0)),
            pl.BlockSpec((2, RB), lambda i: (0, i)),
            pl.BlockSpec((64,), lambda i: (0,)),
        ],
        out_specs=pl.BlockSpec((RB, 64), lambda i: (i, 0)),
        out_shape=jax.ShapeDtypeStruct((NP, 64), f32),
    )(s2.reshape(2, NP, CW), y2, degp2, b2)

    return out[:N]


# speedup vs baseline: 13.8718x; 1.4232x over previous
import functools

import jax
import jax.numpy as jnp
from jax import lax
from jax.experimental import pallas as pl
from jax.experimental.pallas import tpu as pltpu
from jax.experimental.pallas import tpu_sc as plsc

N = 50000
E = 800000
NP = 51200
EP = 819200
CW = 32
BN = 40
BE = 128
NC, NS = 2, 16
RW = NP // (NC * NS)
RT = NP // NS
EPB = EP // BE
ETR = EP // NS // BE
EHR = EP // (NC * NS) // BE
IB = 40

_mesh = plsc.VectorSubcoreMesh(core_axis_name="c", subcore_axis_name="s")


@functools.partial(
    pl.kernel,
    out_type=(
        jax.ShapeDtypeStruct((NP, CW), jnp.float32),
        jax.ShapeDtypeStruct((NP, CW), jnp.float32),
        jax.ShapeDtypeStruct((NP, CW), jnp.float32),
        jax.ShapeDtypeStruct((NC * NP,), jnp.float32),
    ),
    mesh=_mesh,
    compiler_params=pltpu.CompilerParams(use_tc_tiling_on_sc=False),
    scratch_types=[
        pltpu.VMEM((RW // BN, BN), jnp.int32),
        pltpu.VMEM((RW, CW), jnp.float32),
        pltpu.VMEM((EHR, BE), jnp.int32),
        pltpu.VMEM((BE,), jnp.float32),
        pltpu.VMEM_SHARED((NP,), jnp.float32),
        pltpu.SemaphoreType.DMA,
    ],
)
def _embed_deg(cat_h, sub_h, elem_h, dst_h, ones_h, zeros1_h,
               tcat_h, tsub_h, telem_h,
               xc_h, xs_h, xe_h, degp_h,
               idx_v, rows_v, eidx_v, ones_v, deg_acc, sem):
    cid = lax.axis_index("c")
    tid = lax.axis_index("s")
    wid = tid * NC + cid

    row0 = pl.multiple_of(tid * RT, 8)
    pltpu.sync_copy(zeros1_h, deg_acc.at[pl.ds(row0, RT)])
    pltpu.sync_copy(ones_h, ones_v)
    ebase = pl.multiple_of((cid * NS + tid) * EHR, 8)
    pltpu.sync_copy(dst_h.at[pl.ds(ebase, EHR), :], eidx_v)
    plsc.subcore_barrier()

    def dbody(j, c):
        pltpu.sync_copy(ones_v, deg_acc.at[eidx_v.at[j]], add=True)
        return c
    lax.fori_loop(0, EHR, dbody, 0)

    base = pl.multiple_of(wid * RW, 8)
    brow = pl.multiple_of(wid * (RW // BN), 8)
    for tab_h, nidx_h, out_h in ((tcat_h, cat_h, xc_h),
                                 (tsub_h, sub_h, xs_h),
                                 (telem_h, elem_h, xe_h)):
        pltpu.sync_copy(nidx_h.at[pl.ds(brow, RW // BN), :], idx_v)
        cps = [
            pltpu.async_copy(tab_h.at[idx_v.at[j]],
                             rows_v.at[pl.ds(j * BN, BN), :], sem)
            for j in range(RW // BN)
        ]
        for cp in cps:
            cp.wait()
        pltpu.sync_copy(rows_v, out_h.at[pl.ds(base, RW), :])

    plsc.subcore_barrier()
    orow = pl.multiple_of(cid * NP + tid * RT, 8)
    pltpu.sync_copy(deg_acc.at[pl.ds(row0, RT)], degp_h.at[pl.ds(orow, RT)])


def _make_agg(C):
    @functools.partial(
        pl.kernel,
        out_type=jax.ShapeDtypeStruct((C * NP, CW), jnp.float32),
        mesh=_mesh,
        compiler_params=pltpu.CompilerParams(use_tc_tiling_on_sc=False),
        scratch_types=[
            pltpu.VMEM((IB, BE), jnp.int32),
            pltpu.VMEM((IB, BE), jnp.int32),
            pltpu.VMEM((4, BE, CW), jnp.float32),
            pltpu.VMEM_SHARED((NP, CW), jnp.float32),
            pltpu.SemaphoreType.DMA,
        ],
    )
    def agg(y_h, srcadj_h, dst_h, zeros_h, s_h,
            src_v, dst_v, rows_v, acc, sem):
        cid = lax.axis_index("c")
        tid = lax.axis_index("s")
        row0 = pl.multiple_of(tid * RT, 8)
        tb = pl.multiple_of(tid * ETR, 8)

        for k in range(C // 2):
            chunk = cid * (C // 2) + k
            pltpu.sync_copy(zeros_h, acc.at[pl.ds(row0, RT), :])
            plsc.subcore_barrier()

            srow = pl.multiple_of(chunk * EPB + tb, 8)

            def blk_body(blk, c):
                boff = blk * IB
                pltpu.sync_copy(srcadj_h.at[pl.ds(srow + boff, IB), :], src_v)
                pltpu.sync_copy(dst_h.at[pl.ds(tb + boff, IB), :], dst_v)
                descs = {}
                for j in range(3):
                    descs[j] = pltpu.async_copy(
                        y_h.at[src_v.at[j]], rows_v.at[j % 4], sem)
                for j in range(IB):
                    descs[j].wait()
                    if j + 3 < IB:
                        descs[j + 3] = pltpu.async_copy(
                            y_h.at[src_v.at[j + 3]], rows_v.at[(j + 3) % 4],
                            sem)
                    pltpu.sync_copy(rows_v.at[j % 4], acc.at[dst_v.at[j]],
                                    add=True)
                return c
            lax.fori_loop(0, ETR // IB, blk_body, 0)
            plsc.subcore_barrier()

            orow = pl.multiple_of(chunk * NP + tid * RT, 8)
            pltpu.sync_copy(acc.at[pl.ds(row0, RT), :],
                            s_h.at[pl.ds(orow, RT), :])
    return agg


_agg4 = _make_agg(4)
_agg2 = _make_agg(2)

RB = 512


def _y1_body(xc, xs, xe, dg, w1, y1o):
    dinv = lax.rsqrt(dg[0, :] + dg[1, :] + 1.0)
    xw = (jnp.dot(xc[...], w1[0:32, :], preferred_element_type=jnp.float32)
          + jnp.dot(xs[...], w1[32:64, :], preferred_element_type=jnp.float32)
          + jnp.dot(xe[...], w1[64:96, :], preferred_element_type=jnp.float32))
    y = xw * dinv[:, None]
    for c in range(4):
        y1o[c] = y[:, c * CW:(c + 1) * CW]


def _y2_body(s1, y1, dg, b1, w2, y2o):
    dinv = lax.rsqrt(dg[0, :] + dg[1, :] + 1.0)
    acc = jnp.zeros((RB, 64), jnp.float32)
    for c in range(4):
        t = jnp.maximum((s1[c] + y1[c]) * dinv[:, None]
                        + b1[c * CW:(c + 1) * CW][None, :], 0.0)
        acc = acc + jnp.dot(t, w2[c * CW:(c + 1) * CW, :],
                            preferred_element_type=jnp.float32)
    y2 = acc * dinv[:, None]
    y2o[0] = y2[:, 0:CW]
    y2o[1] = y2[:, CW:2 * CW]


def _out_body(s2, y2, dg, b2, o):
    dinv = lax.rsqrt(dg[0, :] + dg[1, :] + 1.0)
    o0 = jnp.maximum((s2[0] + y2[0]) * dinv[:, None] + b2[0:CW][None, :], 0.0)
    o1 = jnp.maximum((s2[1] + y2[1]) * dinv[:, None] + b2[CW:2 * CW][None, :],
                     0.0)
    o[...] = jnp.concatenate([o0, o1], axis=1)


def kernel(cat_idx, sub_idx, elem_idx, edge_index,
           emb_cat, emb_sub, emb_elem, W1, b1, W2, b2):
    f32 = jnp.float32
    cat2 = jnp.pad(cat_idx, (0, NP - N)).reshape(NP // BN, BN)
    sub2 = jnp.pad(sub_idx, (0, NP - N)).reshape(NP // BN, BN)
    elem2 = jnp.pad(elem_idx, (0, NP - N)).reshape(NP // BN, BN)
    src_p = jnp.pad(edge_index[0], (0, EP - E), constant_values=NP - 1)
    dst_p = jnp.pad(edge_index[1], (0, EP - E), constant_values=NP - 1)
    dst2 = dst_p.reshape(EPB, BE)
    srcadj4 = (src_p[None, :]
               + (jnp.arange(4, dtype=jnp.int32) * NP)[:, None]
               ).reshape(4 * EPB, BE)
    srcadj2 = srcadj4[:2 * EPB]
    ones_be = jnp.ones((BE,), f32)
    zeros_rt = jnp.zeros((RT,), f32)
    zeros_zb = jnp.zeros((RT, CW), f32)

    xc, xs, xe, degp = _embed_deg(cat2, sub2, elem2, dst2, ones_be, zeros_rt,
                                  emb_cat, emb_sub, emb_elem)
    degp2 = degp.reshape(2, NP)

    y1 = pl.pallas_call(
        _y1_body,
        grid=(NP // RB,),
        in_specs=[
            pl.BlockSpec((RB, CW), lambda i: (i, 0)),
            pl.BlockSpec((RB, CW), lambda i: (i, 0)),
            pl.BlockSpec((RB, CW), lambda i: (i, 0)),
            pl.BlockSpec((2, RB), lambda i: (0, i)),
            pl.BlockSpec((96, 128), lambda i: (0, 0)),
        ],
        out_specs=pl.BlockSpec((4, RB, CW), lambda i: (0, i, 0)),
        out_shape=jax.ShapeDtypeStruct((4, NP, CW), f32),
    )(xc, xs, xe, degp2, W1)

    s1 = _agg4(y1.reshape(4 * NP, CW), srcadj4, dst2, zeros_zb)

    y2 = pl.pallas_call(
        _y2_body,
        grid=(NP // RB,),
        in_specs=[
            pl.BlockSpec((4, RB, CW), lambda i: (0, i, 0)),
            pl.BlockSpec((4, RB, CW), lambda i: (0, i, 0)),
            pl.BlockSpec((2, RB), lambda i: (0, i)),
            pl.BlockSpec((128,), lambda i: (0,)),
            pl.BlockSpec((128, 64), lambda i: (0, 0)),
        ],
        out_specs=pl.BlockSpec((2, RB, CW), lambda i: (0, i, 0)),
        out_shape=jax.ShapeDtypeStruct((2, NP, CW), f32),
    )(s1.reshape(4, NP, CW), y1, degp2, b1, W2)

    s2 = _agg2(y2.reshape(2 * NP, CW), srcadj2, dst2, zeros_zb)

    out = pl.pallas_call(
        _out_body,
        grid=(NP // RB,),
        in_specs=[
            pl.BlockSpec((2, RB, CW), lambda i: (0, i, 0)),
            pl.BlockSpec((2, RB, CW), lambda i: (0, i, 0)),
            pl.BlockSpec((2, RB), lambda i: (0, i)),
            pl.BlockSpec((64,), lambda i: (0,)),
        ],
        out_specs=pl.BlockSpec((RB, 64), lambda i: (i, 0)),
        out_shape=jax.ShapeDtypeStruct((NP, 64), f32),
    )(s2.reshape(2, NP, CW), y2, degp2, b2)

    return out[:N]
